# Initial kernel scaffold; baseline (speedup 1.0000x reference)
import jax
import jax.numpy as jnp
from jax.experimental import pallas as pl


def kernel(objects_pc, poses, params):
    raise NotImplementedError("write your pallas kernel here")


# speedup vs baseline: 4.0712x; 4.0712x over previous
import functools

import jax
import jax.numpy as jnp
from jax.experimental import pallas as pl
from jax.experimental.pallas import tpu as pltpu

F32 = jnp.float32
HI = jax.lax.Precision.HIGHEST
DEF = jax.lax.Precision.DEFAULT

N_NODE = 61
KNN = 60
N_EDGE = N_NODE * KNN
E_PAD = 4096
TILE = 512
N_TILES = E_PAD // TILE
HID = 512
DOUT = 256


def _pn_body(cmajor, x_ref, W1, b1, Wt1, bt1, Wt2, bt2, Wt3, bt3, Wt4, bt4,
             Wt5, bt5, W2, b2, W3, b3, out_ref):
    dot = functools.partial(jnp.dot, precision=DEF)
    x = x_ref[0]
    if cmajor:
        h0 = jax.lax.dot_general(x, W1[...], (((0,), (0,)), ((), ())),
                                 precision=DEF)
    else:
        h0 = dot(x, W1[...])
    h = jax.nn.relu(h0 + b1[...])
    t = jax.nn.relu(dot(h, Wt1[...]) + bt1[...])
    t = jax.nn.relu(dot(t, Wt2[...]) + bt2[...])
    tm = jnp.max(t, axis=0, keepdims=True)
    t = jax.nn.relu(dot(tm, Wt3[...]) + bt3[...])
    t = jax.nn.relu(dot(t, Wt4[...]) + bt4[...])
    Af = dot(t, Wt5[...]) + bt5[...]
    A = jnp.concatenate([Af[:, i * 64:(i + 1) * 64] for i in range(64)],
                        axis=0)
    r = jax.lax.broadcasted_iota(jnp.int32, (64, 64), 0)
    c = jax.lax.broadcasted_iota(jnp.int32, (64, 64), 1)
    A = A + jnp.where(r == c, 1.0, 0.0).astype(F32)
    h2 = dot(h, A)
    h3 = jax.nn.relu(dot(h2, W2[...]) + b2[...])
    h4 = dot(h3, W3[...]) + b3[...]
    out_ref[0, 0] = jnp.max(h4, axis=0)


def _pallas_pn(xT, p, cmajor, interpret=False):
    B = xT.shape[0]
    C, N = (xT.shape[1], xT.shape[2]) if cmajor else (xT.shape[2], xT.shape[1])
    wnames = ['W1', 'b1', 'Wt1', 'bt1', 'Wt2', 'bt2', 'Wt3', 'bt3', 'Wt4',
              'bt4', 'Wt5', 'bt5', 'W2', 'b2', 'W3', 'b3']
    ws = [p[n] for n in wnames]
    xblock = (1, C, N) if cmajor else (1, N, C)
    in_specs = [pl.BlockSpec(xblock, lambda i: (i, 0, 0))]
    for w in ws:
        in_specs.append(
            pl.BlockSpec(w.shape, (lambda nd: lambda i: (0,) * nd)(w.ndim)))
    return pl.pallas_call(
        functools.partial(_pn_body, cmajor),
        grid=(B,),
        in_specs=in_specs,
        out_specs=pl.BlockSpec((1, 1, 1024), lambda i: (i, 0, 0)),
        out_shape=jax.ShapeDtypeStruct((B, 1, 1024), F32),
        interpret=interpret,
    )(xT, *ws)[:, 0, :]


def _onehot_t(idx_row, tile):
    rows = jax.lax.broadcasted_iota(jnp.int32, (N_NODE, tile), 0)
    return (rows == idx_row[None, :]).astype(F32)


def _gnn_body(X_ref, n2_ref, o_ref,
              W1a0, b1a0, W1b0, b1b0, W2a0, b2a0, W2b0, b2b0,
              W1a1, b1a1, W1b1, b1b1, W2a1, b2a1, W2b1, b2b1,
              W1a2, b1a2, W1b2, b1b2, W2a2, b2a2, W2b2, b2b2,
              W1a3, b1a3, W1b3, b1b3, W2a3, b2a3, W2b3, b2b3,
              W1a4, b1a4, W1b4, b1b4, W2a4, b2a4, W2b4, b2b4,
              ocW1, ocb1, ocW2, ocb2, rcW1, rcb1, rcW2, rcb2,
              obj_ref, rel_ref, pvA, pvB):
    dotD = functools.partial(jnp.dot, precision=DEF)
    dotH = functools.partial(jnp.dot, precision=HI)
    gathT = lambda a, b: jax.lax.dot_general(
        a, b, (((0,), (0,)), ((), ())), precision=HI)
    X = X_ref[0]

    layers = [
        (W1a0, b1a0, W1b0, b1b0, W2a0, b2a0, W2b0, b2b0),
        (W1a1, b1a1, W1b1, b1b1, W2a1, b2a1, W2b1, b2b1),
        (W1a2, b1a2, W1b2, b1b2, W2a2, b2a2, W2b2, b2b2),
        (W1a3, b1a3, W1b3, b1b3, W2a3, b2a3, W2b3, b2b3),
        (W1a4, b1a4, W1b4, b1b4, W2a4, b2a4, W2b4, b2b4),
    ]

    oo = X
    pv_in, pv_out = pvA, pvB
    for li, (W1a, b1a, W1b, b1b, W2a, b2a, W2b, b2b) in enumerate(layers):
        def tile_step(t, carry, li=li, oo=oo, pv_in=pv_in, pv_out=pv_out,
                      W1a=W1a, b1a=b1a, W1b=W1b, b1b=b1b):
            pooled, cnt_o = carry
            et = t * TILE + jax.lax.broadcasted_iota(jnp.int32, (1, TILE), 1)[0]
            s_t = jnp.where(et < N_EDGE, et // KNN, N_NODE)
            o_t = o_ref[0, 0, pl.ds(t * TILE, TILE)]
            St = _onehot_t(s_t, TILE)
            Ot = _onehot_t(o_t, TILE)
            Xs = gathT(St, oo)
            Xo = gathT(Ot, oo)
            if li == 0:
                n2_t = n2_ref[0, 0, pl.ds(t * TILE, TILE)]
                pv = gathT(_onehot_t(n2_t, TILE), oo) - Xs
            else:
                pv = pv_in[pl.ds(t * TILE, TILE), :]
            cur = jnp.concatenate([Xs, pv, Xo], axis=1)
            h = jax.nn.relu(dotD(cur, W1a[...]) + b1a[...])
            hh = dotD(h, W1b[...]) + b1b[...]
            pv_out[pl.ds(t * TILE, TILE), :] = hh[:, HID:HID + DOUT]
            pooled = pooled + dotH(St, hh[:, :HID]) + dotH(Ot, hh[:, HID + DOUT:])
            cnt_o = cnt_o + jnp.sum(Ot, axis=1, keepdims=True)
            return pooled, cnt_o

        pooled, cnt_o = jax.lax.fori_loop(
            0, N_TILES, tile_step,
            (jnp.zeros((N_NODE, HID), F32), jnp.zeros((N_NODE, 1), F32)))
        cnt = cnt_o + 60.0
        pooled = pooled / jnp.maximum(cnt, 1.0)
        g = jax.nn.relu(dotD(pooled, W2a[...]) + b2a[...])
        oo = dotD(g, W2b[...]) + b2b[...]
        pv_in, pv_out = pv_out, pv_in

    op = dotD(jax.nn.relu(dotD(oo, ocW1[...]) + ocb1[...]), ocW2[...]) + ocb2[...]
    obj_ref[0] = op

    def rel_step(t, _, pv_in=pv_in):
        pv = pv_in[pl.ds(t * TILE, TILE), :]
        rp = dotD(jax.nn.relu(dotD(pv, rcW1[...]) + rcb1[...]),
                  rcW2[...]) + rcb2[...]
        rel_ref[0, pl.ds(t * TILE, TILE), :] = rp
        return 0

    jax.lax.fori_loop(0, N_TILES, rel_step, 0)


def _pallas_gnn(Xn, n2_pad, o_pad, params, interpret=False):
    ws = []
    for l in range(5):
        p = params['gconv%d' % l]
        ws += [p['W1a'], p['b1a'].reshape(1, -1), p['W1b'],
               p['b1b'].reshape(1, -1), p['W2a'], p['b2a'].reshape(1, -1),
               p['W2b'], p['b2b'].reshape(1, -1)]
    ws += [params['ocW1'], params['ocb1'].reshape(1, -1),
           params['ocW2'], params['ocb2'].reshape(1, -1),
           params['rcW1'], params['rcb1'].reshape(1, -1),
           params['rcW2'], params['rcb2'].reshape(1, -1)]
    in_specs = [
        pl.BlockSpec((1, N_NODE, 1024), lambda i: (i, 0, 0)),
        pl.BlockSpec((1, 1, E_PAD), lambda i: (i, 0, 0)),
        pl.BlockSpec((1, 1, E_PAD), lambda i: (i, 0, 0)),
    ]
    for w in ws:
        in_specs.append(
            pl.BlockSpec(w.shape, (lambda nd: lambda i: (0,) * nd)(w.ndim)))
    return pl.pallas_call(
        _gnn_body,
        grid=(2,),
        in_specs=in_specs,
        out_specs=[
            pl.BlockSpec((1, N_NODE, 160), lambda i: (i, 0, 0)),
            pl.BlockSpec((1, E_PAD, 27), lambda i: (i, 0, 0)),
        ],
        out_shape=[
            jax.ShapeDtypeStruct((2, N_NODE, 160), F32),
            jax.ShapeDtypeStruct((2, E_PAD, 27), F32),
        ],
        scratch_shapes=[
            pltpu.VMEM((E_PAD, DOUT), F32),
            pltpu.VMEM((E_PAD, DOUT), F32),
        ],
        interpret=interpret,
    )(Xn, n2_pad, o_pad, *ws)


def _forward_impl(objects_pc, poses, params, interpret=False):
    obj_g = _pallas_pn(objects_pc[:60], params['obj_pn'], cmajor=False,
                       interpret=interpret)
    pred_g = _pallas_pn(poses, params['rel_pn'], cmajor=True,
                        interpret=interpret)
    ovT = jnp.broadcast_to(jnp.transpose(obj_g)[None], (2, 1024, 60))
    Xd = jnp.concatenate([ovT, pred_g[:, :, None]], axis=2)
    inner = -2.0 * jnp.einsum('bdn,bdm->bnm', Xd, Xd)
    xx = jnp.sum(Xd * Xd, axis=1, keepdims=True)
    pdist = -xx - inner - jnp.transpose(xx, (0, 2, 1))
    idx = jax.lax.top_k(pdist, KNN)[1]

    n2 = idx.reshape(2, N_EDGE)
    o = jnp.transpose(idx, (0, 2, 1)).reshape(2, N_EDGE)
    pad = ((0, 0), (0, E_PAD - N_EDGE))
    n2_pad = jnp.pad(n2, pad, constant_values=N_NODE).reshape(2, 1, E_PAD)
    o_pad = jnp.pad(o, pad, constant_values=N_NODE).reshape(2, 1, E_PAD)

    Xn = jnp.transpose(Xd, (0, 2, 1))
    obj_preds, rel_pad = _pallas_gnn(Xn, n2_pad, o_pad, params,
                                     interpret=interpret)
    return obj_preds, rel_pad[:, :N_EDGE, :]


def kernel(objects_pc, poses, params):
    return _forward_impl(objects_pc, poses, params)
